# Initial kernel scaffold; baseline (speedup 1.0000x reference)
import jax
import jax.numpy as jnp
from jax.experimental import pallas as pl


def kernel(x, W, train_features):
    raise NotImplementedError("write your pallas kernel here")


# speedup vs baseline: 6.3549x; 6.3549x over previous
import jax
import jax.numpy as jnp
from jax.experimental import pallas as pl
from jax.experimental.pallas import tpu as pltpu

_Q = 1024
_D_IN = 256
_D_FEAT = 128
_N_TRAIN = 100000
_BN = 1000


def _knn_body(x_ref, w_ref, tf_ref, out_ref, feats_ref, m1_ref, m2_ref):
    step = pl.program_id(0)
    nsteps = pl.num_programs(0)

    @pl.when(step == 0)
    def _init():
        feats_ref[...] = jnp.dot(
            x_ref[...], w_ref[...], preferred_element_type=jnp.float32)
        m1_ref[...] = jnp.full((_Q, _BN), jnp.inf, jnp.float32)
        m2_ref[...] = jnp.full((_Q, _BN), jnp.inf, jnp.float32)

    feats = feats_ref[...]
    tf = tf_ref[...]
    ones = jnp.ones((1, _D_FEAT), jnp.float32)
    k_sq = jax.lax.dot_general(
        ones, tf * tf, (((1,), (1,)), ((), ())),
        preferred_element_type=jnp.float32)
    dots = jax.lax.dot_general(
        feats, tf, (((1,), (1,)), ((), ())),
        preferred_element_type=jnp.float32)
    s = k_sq - 2.0 * dots

    r1 = m1_ref[...]
    r2 = m2_ref[...]
    n1 = jnp.minimum(r1, s)
    n2 = jnp.minimum(jnp.maximum(r1, s), r2)
    m1_ref[...] = n1
    m2_ref[...] = n2

    @pl.when(step == nsteps - 1)
    def _fin():
        g1 = jnp.min(n1, axis=1, keepdims=True)
        am = jnp.argmin(n1, axis=1)
        col = jax.lax.broadcasted_iota(jnp.int32, n1.shape, 1)
        at_min = col == am[:, None]
        sec_r1 = jnp.min(jnp.where(at_min, jnp.inf, n1), axis=1, keepdims=True)
        r2_at = jnp.min(jnp.where(at_min, n2, jnp.inf), axis=1, keepdims=True)
        g2 = jnp.minimum(sec_r1, r2_at)
        q_sq = jnp.sum(feats * feats, axis=1, keepdims=True)
        out_ref[...] = g1 + g2 + 2.0 * q_sq


def kernel(x, W, train_features):
    grid = (_N_TRAIN // _BN,)
    out = pl.pallas_call(
        _knn_body,
        grid=grid,
        in_specs=[
            pl.BlockSpec((_Q, _D_IN), lambda i: (0, 0)),
            pl.BlockSpec((_D_IN, _D_FEAT), lambda i: (0, 0)),
            pl.BlockSpec((_BN, _D_FEAT), lambda i: (i, 0)),
        ],
        out_specs=pl.BlockSpec((_Q, 1), lambda i: (0, 0)),
        out_shape=jax.ShapeDtypeStruct((_Q, 1), jnp.float32),
        scratch_shapes=[
            pltpu.VMEM((_Q, _D_FEAT), jnp.float32),
            pltpu.VMEM((_Q, _BN), jnp.float32),
            pltpu.VMEM((_Q, _BN), jnp.float32),
        ],
        compiler_params=pltpu.CompilerParams(
            dimension_semantics=("arbitrary",),
        ),
    )(x, W, train_features)
    return out[:, 0]


# speedup vs baseline: 6.6092x; 1.0400x over previous
import jax
import jax.numpy as jnp
from jax.experimental import pallas as pl
from jax.experimental.pallas import tpu as pltpu

_Q = 1024
_D_IN = 256
_D_FEAT = 128
_N_TRAIN = 100000
_BN = 2000


def _knn_body(x_ref, w_ref, tf_ref, out_ref, feats_ref, m1_ref, m2_ref):
    step = pl.program_id(0)
    nsteps = pl.num_programs(0)

    @pl.when(step == 0)
    def _init():
        feats_ref[...] = jnp.dot(
            x_ref[...], w_ref[...], preferred_element_type=jnp.float32)
        m1_ref[...] = jnp.full((_Q, _BN), jnp.inf, jnp.float32)
        m2_ref[...] = jnp.full((_Q, _BN), jnp.inf, jnp.float32)

    feats = feats_ref[...]
    tf = tf_ref[...]
    ones = jnp.ones((1, _D_FEAT), jnp.float32)
    k_sq = jax.lax.dot_general(
        ones, tf * tf, (((1,), (1,)), ((), ())),
        preferred_element_type=jnp.float32)
    dots = jax.lax.dot_general(
        feats, tf, (((1,), (1,)), ((), ())),
        preferred_element_type=jnp.float32)
    s = k_sq - 2.0 * dots

    r1 = m1_ref[...]
    r2 = m2_ref[...]
    n1 = jnp.minimum(r1, s)
    n2 = jnp.minimum(jnp.maximum(r1, s), r2)
    m1_ref[...] = n1
    m2_ref[...] = n2

    @pl.when(step == nsteps - 1)
    def _fin():
        g1 = jnp.min(n1, axis=1, keepdims=True)
        am = jnp.argmin(n1, axis=1)
        col = jax.lax.broadcasted_iota(jnp.int32, n1.shape, 1)
        at_min = col == am[:, None]
        sec_r1 = jnp.min(jnp.where(at_min, jnp.inf, n1), axis=1, keepdims=True)
        r2_at = jnp.min(jnp.where(at_min, n2, jnp.inf), axis=1, keepdims=True)
        g2 = jnp.minimum(sec_r1, r2_at)
        q_sq = jnp.sum(feats * feats, axis=1, keepdims=True)
        out_ref[...] = g1 + g2 + 2.0 * q_sq


def kernel(x, W, train_features):
    grid = (_N_TRAIN // _BN,)
    out = pl.pallas_call(
        _knn_body,
        grid=grid,
        in_specs=[
            pl.BlockSpec((_Q, _D_IN), lambda i: (0, 0)),
            pl.BlockSpec((_D_IN, _D_FEAT), lambda i: (0, 0)),
            pl.BlockSpec((_BN, _D_FEAT), lambda i: (i, 0)),
        ],
        out_specs=pl.BlockSpec((_Q, 1), lambda i: (0, 0)),
        out_shape=jax.ShapeDtypeStruct((_Q, 1), jnp.float32),
        scratch_shapes=[
            pltpu.VMEM((_Q, _D_FEAT), jnp.float32),
            pltpu.VMEM((_Q, _BN), jnp.float32),
            pltpu.VMEM((_Q, _BN), jnp.float32),
        ],
        compiler_params=pltpu.CompilerParams(
            dimension_semantics=("arbitrary",),
        ),
    )(x, W, train_features)
    return out[:, 0]
